# Initial kernel scaffold; baseline (speedup 1.0000x reference)
import jax
import jax.numpy as jnp
from jax.experimental import pallas as pl


def kernel(x_features, xyz_coords, W1, gamma1, beta1, W2, gamma2, beta2):
    raise NotImplementedError("write your pallas kernel here")


# speedup vs baseline: 13.8050x; 13.8050x over previous
import functools

import jax
import jax.numpy as jnp
from jax import lax
from jax.experimental import pallas as pl
from jax.experimental.pallas import tpu as pltpu
from jax.experimental.pallas import tpu_sc as plsc

KNN = 20
NPTS = 4096
NBATCH = 2
DF = 256
CH = 32
M = NBATCH * NPTS

ROWS = 256

SC_NC = 2
SC_NS = 16
SC_NW = SC_NC * SC_NS
PW = M // SC_NW
PC = 32
NCHUNK = PW // PC
IDX_PER_CHUNK = PC * KNN
GATHERS_PER_CHUNK = IDX_PER_CHUNK // 128
GROW = 128


def _knn_proj_kernel(xyz_nd_ref, xyz_dn_ref, x_ref, w1a_ref, w1b_ref,
                     idx_ref, y_ref, w_ref):
    b = pl.program_id(0)
    x = x_ref[0]
    y = jnp.dot(x, w1a_ref[...], preferred_element_type=jnp.float32)
    z = jnp.dot(x, w1b_ref[...], preferred_element_type=jnp.float32)
    y_ref[0] = jnp.concatenate(
        [y, jnp.zeros((ROWS, GROW - CH), jnp.float32)], axis=1)
    w_ref[0] = y - z
    rows = xyz_nd_ref[0]
    alln = xyz_dn_ref[0]
    sqrow = jnp.sum(rows * rows, axis=1, keepdims=True)
    sqall = jnp.sum(alln * alln, axis=0, keepdims=True)
    cross = jnp.dot(rows, alln, preferred_element_type=jnp.float32)
    d = -2.0 * cross + sqrow + sqall
    cols = lax.broadcasted_iota(jnp.int32, (ROWS, NPTS), 1)
    big = jnp.int32(2**30)
    for k in range(KNN):
        m = jnp.min(d, axis=1, keepdims=True)
        am = jnp.min(jnp.where(d == m, cols, big), axis=1, keepdims=True)
        idx_ref[0, :, k:k + 1] = am + b * NPTS
        d = jnp.where(cols == am, jnp.float32(jnp.inf), d)


def _knn_proj(xyz_nd, xyz_dn, x, w1a_t, w1b_t):
    nb = NPTS // ROWS
    return pl.pallas_call(
        _knn_proj_kernel,
        grid=(NBATCH, nb),
        in_specs=[
            pl.BlockSpec((1, ROWS, 3), lambda b, i: (b, i, 0)),
            pl.BlockSpec((1, 3, NPTS), lambda b, i: (b, 0, 0)),
            pl.BlockSpec((1, ROWS, DF), lambda b, i: (b, i, 0)),
            pl.BlockSpec((DF, CH), lambda b, i: (0, 0)),
            pl.BlockSpec((DF, CH), lambda b, i: (0, 0)),
        ],
        out_specs=[
            pl.BlockSpec((1, ROWS, KNN), lambda b, i: (b, i, 0)),
            pl.BlockSpec((1, ROWS, GROW), lambda b, i: (b, i, 0)),
            pl.BlockSpec((1, ROWS, CH), lambda b, i: (b, i, 0)),
        ],
        out_shape=[
            jax.ShapeDtypeStruct((NBATCH, NPTS, KNN), jnp.int32),
            jax.ShapeDtypeStruct((NBATCH, NPTS, GROW), jnp.float32),
            jax.ShapeDtypeStruct((NBATCH, NPTS, CH), jnp.float32),
        ],
    )(xyz_nd, xyz_dn, x, w1a_t, w1b_t)


def _gather_reduce_body(y_hbm, idx_hbm, sum_hbm, sq_hbm, mx_hbm,
                        idx_v, rows_v, sum_v, sq_v, mx_v, sem):
    cid = lax.axis_index("c")
    sid = lax.axis_index("s")
    wid = sid * SC_NC + cid
    pltpu.sync_copy(idx_hbm.at[wid], idx_v)

    def chunk_body(c, carry):
        base_pt = wid * PW + c * PC
        cps = []
        for j in range(GATHERS_PER_CHUNK):
            cps.append(pltpu.async_copy(
                y_hbm.at[idx_v.at[c * GATHERS_PER_CHUNK + j]],
                rows_v.at[pl.ds(j * 128, 128)], sem))
        for cp in cps:
            cp.wait()

        def point_body(p, carry2):
            base = p * KNN
            for h in range(2):
                lanes = pl.ds(h * 16, 16)
                v = rows_v[base, lanes]
                s = v
                q = v * v
                mx = v
                for k in range(1, KNN):
                    v = rows_v[base + k, lanes]
                    s = s + v
                    q = q + v * v
                    mx = jnp.maximum(mx, v)
                sum_v[p, lanes] = s
                sq_v[p, lanes] = q
                mx_v[p, lanes] = mx
            return carry2

        lax.fori_loop(0, PC, point_body, 0)
        pltpu.sync_copy(sum_v, sum_hbm.at[pl.ds(base_pt, PC)])
        pltpu.sync_copy(sq_v, sq_hbm.at[pl.ds(base_pt, PC)])
        pltpu.sync_copy(mx_v, mx_hbm.at[pl.ds(base_pt, PC)])
        return carry

    lax.fori_loop(0, NCHUNK, chunk_body, 0)


@functools.cache
def _gather_reduce():
    return pl.kernel(
        _gather_reduce_body,
        mesh=plsc.VectorSubcoreMesh(core_axis_name="c", subcore_axis_name="s",
                                    num_cores=SC_NC),
        out_type=[
            jax.ShapeDtypeStruct((M, CH), jnp.float32),
            jax.ShapeDtypeStruct((M, CH), jnp.float32),
            jax.ShapeDtypeStruct((M, CH), jnp.float32),
        ],
        scratch_types=[
            pltpu.VMEM((PW * KNN // 128, 128), jnp.int32),
            pltpu.VMEM((IDX_PER_CHUNK, GROW), jnp.float32),
            pltpu.VMEM((PC, CH), jnp.float32),
            pltpu.VMEM((PC, CH), jnp.float32),
            pltpu.VMEM((PC, CH), jnp.float32),
            pltpu.SemaphoreType.DMA,
        ],
    )


def _final_kernel(w_ref, sg_ref, sq_ref, mx_ref, w2t_ref,
                  g1_ref, b1_ref, g2_ref, b2_ref, out_ref):
    w = w_ref[...]
    sg = sg_ref[...]
    sq = sq_ref[...]
    mx = mx_ref[...]
    kf = jnp.float32(KNN)
    cnt1 = jnp.float32(M * KNN)
    s1 = jnp.sum(sg - kf * w, axis=0, keepdims=True)
    mean1 = s1 / cnt1
    s2 = jnp.sum(sq - 2.0 * w * sg + kf * (w * w), axis=0, keepdims=True)
    var1 = s2 / cnt1 - mean1 * mean1
    inv1 = g1_ref[...] / jnp.sqrt(var1 + 1e-5)
    hmax = mx - w
    a = (hmax - mean1) * inv1 + b1_ref[...]
    a = jnp.where(a >= 0, a, 0.2 * a)
    o = jnp.dot(a, w2t_ref[...], preferred_element_type=jnp.float32)
    cnt2 = jnp.float32(M)
    mean2 = jnp.sum(o, axis=0, keepdims=True) / cnt2
    var2 = jnp.sum(o * o, axis=0, keepdims=True) / cnt2 - mean2 * mean2
    t = (o - mean2) * (g2_ref[...] / jnp.sqrt(var2 + 1e-5)) + b2_ref[...]
    out_ref[...] = jnp.where(t >= 0, t, 0.2 * t)


def _final(w2arr, sg, sq, mx, w2t, g1, b1, g2, b2):
    return pl.pallas_call(
        _final_kernel,
        out_shape=jax.ShapeDtypeStruct((M, DF), jnp.float32),
    )(w2arr, sg, sq, mx, w2t, g1, b1, g2, b2)


def kernel(x_features, xyz_coords, W1, gamma1, beta1, W2, gamma2, beta2):
    xyz_nd = xyz_coords
    xyz_dn = jnp.transpose(xyz_coords, (0, 2, 1))
    w1a_t = jnp.transpose(W1[:, :DF])
    w1b_t = jnp.transpose(W1[:, DF:])
    idx, y, w = _knn_proj(xyz_nd, xyz_dn, x_features, w1a_t, w1b_t)
    y2 = y.reshape(M, GROW)
    w2arr = w.reshape(M, CH)
    idx2d = idx.reshape(SC_NW, PW * KNN // 128, 128)
    sg, sq, mx = _gather_reduce()(y2, idx2d)
    out = _final(w2arr, sg, sq, mx, jnp.transpose(W2),
                 gamma1.reshape(1, CH), beta1.reshape(1, CH),
                 gamma2.reshape(1, DF), beta2.reshape(1, DF))
    return out.reshape(NBATCH, NPTS, DF)


# speedup vs baseline: 16.0978x; 1.1661x over previous
import functools

import jax
import jax.numpy as jnp
from jax import lax
from jax.experimental import pallas as pl
from jax.experimental.pallas import tpu as pltpu
from jax.experimental.pallas import tpu_sc as plsc

KNN = 20
NPTS = 4096
NBATCH = 2
DF = 256
CH = 32
M = NBATCH * NPTS

ROWS = 512

SC_NC = 2
SC_NS = 16
SC_NW = SC_NC * SC_NS
PW = NPTS // SC_NW
PC = 32
NCHUNK = PW // PC
IDX_PER_CHUNK = PC * KNN
GATHERS_PER_CHUNK = IDX_PER_CHUNK // 128
IDX_ROWS = PW * KNN // 128
GROW = 128


def _knn_proj_kernel(xyz_nd_ref, xyz_dn_ref, x_ref, w1a_ref, w1b_ref,
                     idx_ref, y_ref, w_ref):
    x = x_ref[...]
    y = jnp.dot(x, w1a_ref[...], preferred_element_type=jnp.float32)
    z = jnp.dot(x, w1b_ref[...], preferred_element_type=jnp.float32)
    y_ref[...] = jnp.concatenate(
        [y, jnp.zeros((ROWS, GROW - CH), jnp.float32)], axis=1)
    w_ref[...] = y - z
    rows = xyz_nd_ref[...]
    alln = xyz_dn_ref[...]
    sqrow = jnp.sum(rows * rows, axis=1, keepdims=True)
    sqall = jnp.sum(alln * alln, axis=0, keepdims=True)
    cross = jnp.dot(rows, alln, preferred_element_type=jnp.float32)
    d = -2.0 * cross + sqrow + sqall
    cols = lax.broadcasted_iota(jnp.int32, (ROWS, NPTS), 1).astype(jnp.float32)
    big = jnp.float32(1e9)
    for k in range(KNN):
        m = jnp.min(d, axis=1, keepdims=True)
        am = jnp.min(jnp.where(d == m, cols, big), axis=1, keepdims=True)
        idx_ref[:, k:k + 1] = am.astype(jnp.int32)
        d = jnp.where(cols == am, jnp.float32(jnp.inf), d)


def _knn_proj(xyz_nd, xyz_dn, x, w1a_t, w1b_t):
    nb = NPTS // ROWS
    return pl.pallas_call(
        _knn_proj_kernel,
        grid=(nb,),
        in_specs=[
            pl.BlockSpec((ROWS, 3), lambda i: (i, 0)),
            pl.BlockSpec((3, NPTS), lambda i: (0, 0)),
            pl.BlockSpec((ROWS, DF), lambda i: (i, 0)),
            pl.BlockSpec((DF, CH), lambda i: (0, 0)),
            pl.BlockSpec((DF, CH), lambda i: (0, 0)),
        ],
        out_specs=[
            pl.BlockSpec((ROWS, KNN), lambda i: (i, 0)),
            pl.BlockSpec((ROWS, GROW), lambda i: (i, 0)),
            pl.BlockSpec((ROWS, CH), lambda i: (i, 0)),
        ],
        out_shape=[
            jax.ShapeDtypeStruct((NPTS, KNN), jnp.int32),
            jax.ShapeDtypeStruct((NPTS, GROW), jnp.float32),
            jax.ShapeDtypeStruct((NPTS, CH), jnp.float32),
        ],
    )(xyz_nd, xyz_dn, x, w1a_t, w1b_t)


def _gather_reduce_body(y_hbm, idx_hbm, sum_hbm, sq_hbm, mx_hbm,
                        idx_v, rows_v, sum_v, sq_v, mx_v, sem):
    cid = lax.axis_index("c")
    sid = lax.axis_index("s")
    wid = sid * SC_NC + cid
    pltpu.sync_copy(idx_hbm.at[wid], idx_v)

    def chunk_body(c, carry):
        base_pt = wid * PW + c * PC
        cps = []
        for j in range(GATHERS_PER_CHUNK):
            cps.append(pltpu.async_copy(
                y_hbm.at[idx_v.at[c * GATHERS_PER_CHUNK + j]],
                rows_v.at[pl.ds(j * 128, 128)], sem))
        for cp in cps:
            cp.wait()

        def point_body(p, carry2):
            base = p * KNN
            for h in range(2):
                lanes = pl.ds(h * 16, 16)
                v = rows_v[base, lanes]
                s = v
                q = v * v
                mx = v
                for k in range(1, KNN):
                    v = rows_v[base + k, lanes]
                    s = s + v
                    q = q + v * v
                    mx = jnp.maximum(mx, v)
                sum_v[p, lanes] = s
                sq_v[p, lanes] = q
                mx_v[p, lanes] = mx
            return carry2

        lax.fori_loop(0, PC, point_body, 0)
        pltpu.sync_copy(sum_v, sum_hbm.at[pl.ds(base_pt, PC)])
        pltpu.sync_copy(sq_v, sq_hbm.at[pl.ds(base_pt, PC)])
        pltpu.sync_copy(mx_v, mx_hbm.at[pl.ds(base_pt, PC)])
        return carry

    lax.fori_loop(0, NCHUNK, chunk_body, 0)


@functools.cache
def _gather_reduce():
    return pl.kernel(
        _gather_reduce_body,
        mesh=plsc.VectorSubcoreMesh(core_axis_name="c", subcore_axis_name="s",
                                    num_cores=SC_NC),
        out_type=[
            jax.ShapeDtypeStruct((NPTS, CH), jnp.float32),
            jax.ShapeDtypeStruct((NPTS, CH), jnp.float32),
            jax.ShapeDtypeStruct((NPTS, CH), jnp.float32),
        ],
        scratch_types=[
            pltpu.VMEM((IDX_ROWS, 128), jnp.int32),
            pltpu.VMEM((IDX_PER_CHUNK, GROW), jnp.float32),
            pltpu.VMEM((PC, CH), jnp.float32),
            pltpu.VMEM((PC, CH), jnp.float32),
            pltpu.VMEM((PC, CH), jnp.float32),
            pltpu.SemaphoreType.DMA,
        ],
    )


def _final_kernel(w0_ref, w1_ref, sg0_ref, sg1_ref, sq0_ref, sq1_ref,
                  mx0_ref, mx1_ref, w2t_ref,
                  g1_ref, b1_ref, g2_ref, b2_ref, out_ref):
    w = jnp.concatenate([w0_ref[...], w1_ref[...]], axis=0)
    sg = jnp.concatenate([sg0_ref[...], sg1_ref[...]], axis=0)
    sq = jnp.concatenate([sq0_ref[...], sq1_ref[...]], axis=0)
    mx = jnp.concatenate([mx0_ref[...], mx1_ref[...]], axis=0)
    kf = jnp.float32(KNN)
    cnt1 = jnp.float32(M * KNN)
    s1 = jnp.sum(sg - kf * w, axis=0, keepdims=True)
    mean1 = s1 / cnt1
    s2 = jnp.sum(sq - 2.0 * w * sg + kf * (w * w), axis=0, keepdims=True)
    var1 = s2 / cnt1 - mean1 * mean1
    inv1 = g1_ref[...] / jnp.sqrt(var1 + 1e-5)
    hmax = mx - w
    a = (hmax - mean1) * inv1 + b1_ref[...]
    a = jnp.where(a >= 0, a, 0.2 * a)
    o = jnp.dot(a, w2t_ref[...], preferred_element_type=jnp.float32)
    cnt2 = jnp.float32(M)
    mean2 = jnp.sum(o, axis=0, keepdims=True) / cnt2
    var2 = jnp.sum(o * o, axis=0, keepdims=True) / cnt2 - mean2 * mean2
    t = (o - mean2) * (g2_ref[...] / jnp.sqrt(var2 + 1e-5)) + b2_ref[...]
    out_ref[...] = jnp.where(t >= 0, t, 0.2 * t)


def _final(args):
    return pl.pallas_call(
        _final_kernel,
        out_shape=jax.ShapeDtypeStruct((M, DF), jnp.float32),
    )(*args)


def kernel(x_features, xyz_coords, W1, gamma1, beta1, W2, gamma2, beta2):
    xyz_dn = jnp.transpose(xyz_coords, (0, 2, 1))
    w1a_t = jnp.transpose(W1[:, :DF])
    w1b_t = jnp.transpose(W1[:, DF:])
    ws, sgs, sqs, mxs = [], [], [], []
    for b in range(NBATCH):
        idx, y, w = _knn_proj(xyz_coords[b], xyz_dn[b], x_features[b],
                              w1a_t, w1b_t)
        idx3d = idx.reshape(SC_NW, IDX_ROWS, 128)
        sg, sq, mx = _gather_reduce()(y, idx3d)
        ws.append(w)
        sgs.append(sg)
        sqs.append(sq)
        mxs.append(mx)
    out = _final(ws + sgs + sqs + mxs
                 + [jnp.transpose(W2),
                    gamma1.reshape(1, CH), beta1.reshape(1, CH),
                    gamma2.reshape(1, DF), beta2.reshape(1, DF)])
    return out.reshape(NBATCH, NPTS, DF)
